# Initial kernel scaffold; baseline (speedup 1.0000x reference)
import jax
import jax.numpy as jnp
from jax.experimental import pallas as pl


def kernel(x, edge_index, W1_0, b1_0, g_0, be_0, W2_0, b2_0, W1_1, b1_1, g_1, be_1, W2_1, b2_1):
    raise NotImplementedError("write your pallas kernel here")


# speedup vs baseline: 3.0253x; 3.0253x over previous
import functools

import jax
import jax.numpy as jnp
from jax import lax
from jax.experimental import pallas as pl
from jax.experimental.pallas import tpu as pltpu
from jax.experimental.pallas import tpu_sc as plsc

_NC = 2
_NS = 16
_CH = 128
_L = 16


def _half_nodes(n):
    return (n + 2 * 128 - 1) // (2 * 128) * 128


def _half_acc(n):
    hn = _half_nodes(n)
    return (hn + 1 + 127) // 128 * 128


@functools.lru_cache(maxsize=None)
def _make_agg(n, w, e_pad):
    hn = _half_nodes(n)
    ha = _half_acc(n)
    zr = ha // _NS
    ept = e_pad // _NS
    t_steps = ept // _CH
    mesh = plsc.VectorSubcoreMesh(core_axis_name="c", subcore_axis_name="s")

    @functools.partial(
        pl.kernel,
        out_type=jax.ShapeDtypeStruct((_NC * ha, w), jnp.float32),
        mesh=mesh,
        scratch_types=[
            pltpu.VMEM((_CH,), jnp.int32),
            pltpu.VMEM((_CH,), jnp.int32),
            pltpu.VMEM((_CH, w), jnp.float32),
            pltpu.VMEM((zr, w), jnp.float32),
            pltpu.VMEM_SHARED((ha, w), jnp.float32),
            pltpu.SemaphoreType.DMA,
        ],
        compiler_params=pltpu.CompilerParams(use_tc_tiling_on_sc=False),
    )
    def agg(xp_hbm, src_hbm, dst_hbm, out_hbm, src_v, dst_v, rows_v, zer_v,
            acc_sh, sem):
        c = lax.axis_index("c")
        s = lax.axis_index("s")

        zvec = jnp.zeros((_L,), jnp.float32)
        cpr = w // _L

        def zbody(i, carry):
            zer_v[i // cpr, pl.ds((i % cpr) * _L, _L)] = zvec
            return carry

        lax.fori_loop(0, zr * cpr, zbody, 0)
        pltpu.sync_copy(zer_v, acc_sh.at[pl.ds(s * zr, zr)])
        plsc.subcore_barrier()

        base_node = jnp.broadcast_to(c * hn, (_L,)).astype(jnp.int32)
        trash = jnp.broadcast_to(jnp.int32(hn), (_L,))

        def ebody(t, carry):
            base = s * ept + t * _CH
            pltpu.sync_copy(src_hbm.at[pl.ds(base, _CH)], src_v)
            pltpu.sync_copy(dst_hbm.at[pl.ds(base, _CH)], dst_v)

            def mbody(j, carry2):
                v = dst_v[pl.ds(j * _L, _L)]
                rel = v - base_node
                ok = (rel >= 0) & (rel < hn)
                dst_v[pl.ds(j * _L, _L)] = jnp.where(ok, rel, trash)
                return carry2

            lax.fori_loop(0, _CH // _L, mbody, 0)
            pltpu.async_copy(xp_hbm.at[src_v], rows_v, sem).wait()
            pltpu.sync_copy(rows_v, acc_sh.at[dst_v], add=True)
            return carry

        lax.fori_loop(0, t_steps, ebody, 0)
        plsc.subcore_barrier()

        pltpu.sync_copy(acc_sh.at[pl.ds(s * zr, zr)],
                        out_hbm.at[pl.ds(c * ha + s * zr, zr)])

    return agg


@functools.lru_cache(maxsize=None)
def _make_mlp(n, w, wx, padded_out):
    hn = _half_nodes(n)
    ha = _half_acc(n)
    d = 128

    def body(parts_ref, x_ref, w1_ref, b1_ref, g_ref, be_ref, w2_ref, b2_ref,
             out_ref):
        top = parts_ref[0:hn, :]
        bot = parts_ref[ha:ha + (n - hn), :]
        agg = jnp.concatenate([top, bot], axis=0)
        ssum = agg[:, 0:d]
        cnt = jnp.sum(agg[:, d:w], axis=1, keepdims=True)
        mean = ssum / jnp.maximum(cnt, 1.0)
        x = x_ref[...]
        if wx > d:
            x = x[:, 0:d]
        h = x + mean
        z = lax.dot_general(h, w1_ref[...], (((1,), (1,)), ((), ())),
                            preferred_element_type=jnp.float32)
        z = z + b1_ref[...][None, :]
        mu = jnp.mean(z, axis=0, keepdims=True)
        zc = z - mu
        var = jnp.mean(zc * zc, axis=0, keepdims=True)
        zn = zc * (g_ref[...][None, :] * lax.rsqrt(var + 1e-5))
        zn = zn + be_ref[...][None, :]
        act = jnp.maximum(zn, 0.0)
        o = lax.dot_general(act, w2_ref[...], (((1,), (1,)), ((), ())),
                            preferred_element_type=jnp.float32)
        o = o + b2_ref[...][None, :]
        if padded_out:
            o = jnp.maximum(o, 0.0)
            lane = lax.broadcasted_iota(jnp.int32, (n, w - d), 1)
            pad = jnp.where(lane == 0, 1.0, 0.0)
            out_ref[...] = jnp.concatenate([o, pad], axis=1)
        else:
            out_ref[...] = o

    out_w = w if padded_out else d
    return pl.pallas_call(
        body,
        out_shape=jax.ShapeDtypeStruct((n, out_w), jnp.float32),
    )


def kernel(x, edge_index, W1_0, b1_0, g_0, be_0, W2_0, b2_0,
           W1_1, b1_1, g_1, be_1, W2_1, b2_1):
    n, d = x.shape
    e = edge_index.shape[1]
    w = d + 16
    ept = ((e + _NS - 1) // _NS + _CH - 1) // _CH * _CH
    e_pad = ept * _NS
    pad_e = e_pad - e

    src_p = jnp.concatenate(
        [edge_index[0].astype(jnp.int32), jnp.zeros((pad_e,), jnp.int32)])
    dst_p = jnp.concatenate(
        [edge_index[1].astype(jnp.int32), jnp.full((pad_e,), n, jnp.int32)])
    onescol = jnp.concatenate(
        [jnp.ones((n, 1), jnp.float32), jnp.zeros((n, w - d - 1), jnp.float32)],
        axis=1)
    xp = jnp.concatenate([x.astype(jnp.float32), onescol], axis=1)

    agg = _make_agg(n, w, e_pad)
    parts1 = agg(xp, src_p, dst_p)
    hp = _make_mlp(n, w, d, True)(parts1, x, W1_0, b1_0, g_0, be_0, W2_0, b2_0)
    parts2 = agg(hp, src_p, dst_p)
    out = _make_mlp(n, w, w, False)(parts2, hp, W1_1, b1_1, g_1, be_1, W2_1,
                                    b2_1)
    return out
